# baseline (speedup 1.0000x reference)
import jax
import jax.numpy as jnp
from jax.experimental import pallas as pl
from jax.experimental.pallas import tpu as pltpu

LENS_A = [1500, 900, 2100, 1100]
LENS_B = [500, 1100, 300, 900]
D = 256
N_ORIG = sum(LENS_A) + sum(LENS_B)
N = 8448

_ORDERED_LENS = [LENS_A[0], LENS_B[0], LENS_A[1], LENS_B[1],
                 LENS_A[2], LENS_B[2], LENS_A[3], LENS_B[3]]
_OFFSETS = []
_off = 0
for _l in _ORDERED_LENS:
    _OFFSETS.append(_off)
    _off += _l

_SEG_STARTS = [0, 2000, 4000, 6400]

TILE_R = 384
NTILES = N // TILE_R
NBUF = 4


def _mask_tile(t):
    q = jax.lax.broadcasted_iota(jnp.int32, (TILE_R, 1), 0) + t * TILE_R
    k = jax.lax.broadcasted_iota(jnp.int32, (1, N), 1)

    def seg_id(p):
        s = jnp.zeros(p.shape, jnp.int32)
        for b in _SEG_STARTS[1:]:
            s = s + (p >= b).astype(jnp.int32)
        return s

    m = (seg_id(q) == seg_id(k)) & (q < N_ORIG) & (k < N_ORIG)
    return m.astype(jnp.int32)


def _fused_kernel(a0, a1, a2, a3, b0, b1, b2, b3,
                  packed_out, mask_out, mbuf, zbuf, in_sems, msk_sems):
    ins = [a0, b0, a1, b1, a2, b2, a3, b3]
    for i, (ref, off, l) in enumerate(zip(ins, _OFFSETS, _ORDERED_LENS)):
        pltpu.make_async_copy(
            ref, packed_out.at[2 * off:2 * (off + l), :], in_sems.at[i]
        ).start()
    zbuf[...] = jnp.zeros((2 * (N - N_ORIG), 128), jnp.float32)
    pltpu.make_async_copy(
        zbuf, packed_out.at[2 * N_ORIG:2 * N, :], in_sems.at[8]
    ).start()

    for t in range(NTILES):
        slot = t % NBUF
        if t >= NBUF:
            pltpu.make_async_copy(
                mbuf.at[slot],
                mask_out.at[pl.ds((t - NBUF) * TILE_R, TILE_R), :],
                msk_sems.at[slot],
            ).wait()
        mbuf[slot] = _mask_tile(t)
        pltpu.make_async_copy(
            mbuf.at[slot],
            mask_out.at[pl.ds(t * TILE_R, TILE_R), :],
            msk_sems.at[slot],
        ).start()
    for t in range(max(NTILES - NBUF, 0), NTILES):
        slot = t % NBUF
        pltpu.make_async_copy(
            mbuf.at[slot],
            mask_out.at[pl.ds(t * TILE_R, TILE_R), :],
            msk_sems.at[slot],
        ).wait()
    for i, (ref, off, l) in enumerate(zip(ins, _OFFSETS, _ORDERED_LENS)):
        pltpu.make_async_copy(
            ref, packed_out.at[2 * off:2 * (off + l), :], in_sems.at[i]
        ).wait()
    pltpu.make_async_copy(
        zbuf, packed_out.at[2 * N_ORIG:2 * N, :], in_sems.at[8]
    ).wait()


def kernel(a0, a1, a2, a3, b0, b1, b2, b3):
    views = [jnp.reshape(x, (2 * x.shape[1], 128))
             for x in (a0, a1, a2, a3, b0, b1, b2, b3)]
    packed2d, mask = pl.pallas_call(
        _fused_kernel,
        in_specs=[pl.BlockSpec(memory_space=pl.ANY)] * 8,
        out_specs=(
            pl.BlockSpec(memory_space=pl.ANY),
            pl.BlockSpec(memory_space=pl.ANY),
        ),
        out_shape=(
            jax.ShapeDtypeStruct((2 * N, 128), jnp.float32),
            jax.ShapeDtypeStruct((N, N), jnp.bool_),
        ),
        scratch_shapes=[
            pltpu.VMEM((NBUF, TILE_R, N), jnp.int32),
            pltpu.VMEM((2 * (N - N_ORIG), 128), jnp.float32),
            pltpu.SemaphoreType.DMA((9,)),
            pltpu.SemaphoreType.DMA((NBUF,)),
        ],
    )(*views)
    return jnp.reshape(packed2d, (1, N, D)), mask
